# Initial kernel scaffold; baseline (speedup 1.0000x reference)
import jax
import jax.numpy as jnp
from jax.experimental import pallas as pl


def kernel(x, table, W, b):
    raise NotImplementedError("write your pallas kernel here")


# speedup vs baseline: 26.0002x; 26.0002x over previous
import functools

import jax
import jax.numpy as jnp
from jax import lax
from jax.experimental import pallas as pl
from jax.experimental.pallas import tpu as pltpu
from jax.experimental.pallas import tpu_sc as plsc

B = 16384
HIST = 50
D = 16

NC = 2
NS = 16
NW = NC * NS
BPW = B // NW
CB = 64
NCHUNK = BPW // CB
ROWS = CB * HIST
GN = 128
NG = ROWS // GN


def _sc_partials_body(x_hbm, w_hbm, tab_hbm, out_hbm, w_v, idx_v, rows_v,
                      o_v, sem):
    cid = lax.axis_index("c")
    sid = lax.axis_index("s")
    wid = sid * NC + cid

    pltpu.sync_copy(w_hbm, w_v)

    def chunk_body(chunk, _):
        pltpu.sync_copy(x_hbm.at[wid, chunk], idx_v)
        copies = []
        for j in range(NG):
            copies.append(pltpu.async_copy(
                tab_hbm.at[idx_v.at[j]], rows_v.at[pl.ds(j * GN, GN)], sem))
        for c in copies:
            c.wait()

        def b_body(bb, _):
            base = bb * HIST
            acc0 = jnp.zeros((D,), jnp.float32)
            acc1 = jnp.zeros((D,), jnp.float32)
            acc2 = jnp.zeros((D,), jnp.float32)
            for h in range(HIST):
                r = rows_v[base + h]
                acc0 = acc0 + r * w_v[h]
                acc1 = acc1 + r * w_v[HIST + h]
                acc2 = acc2 + r * w_v[2 * HIST + h]
            o_v[0, bb] = acc0
            o_v[1, bb] = acc1
            o_v[2, bb] = acc2
            return ()

        lax.fori_loop(0, CB, b_body, (), unroll=False)
        out_base = wid * BPW + chunk * CB
        for c in range(3):
            pltpu.sync_copy(o_v.at[c], out_hbm.at[c, pl.ds(out_base, CB)])
        return ()

    lax.fori_loop(0, NCHUNK, chunk_body, (), unroll=False)


def _tc_finish_body(p0_ref, p1_ref, p2_ref, b_ref, o_ref):
    s0 = jnp.sum(p0_ref[...], axis=1, keepdims=True)
    s1 = jnp.sum(p1_ref[...], axis=1, keepdims=True)
    s2 = jnp.sum(p2_ref[...], axis=1, keepdims=True)
    z = jnp.concatenate([s0, s1, s2], axis=1) + b_ref[...]
    m = jnp.max(z, axis=1, keepdims=True)
    e = jnp.exp(z - m)
    lse = jnp.log(jnp.sum(e, axis=1, keepdims=True))
    o_ref[...] = z - m - lse


@jax.jit
def kernel(x, table, W, b):
    x_r = x.astype(jnp.int32).reshape(NW, NCHUNK, NG, GN)
    w_r = W.astype(jnp.float32).reshape(3 * HIST, D)

    mesh = plsc.VectorSubcoreMesh(core_axis_name="c", subcore_axis_name="s")
    sc_fn = functools.partial(
        pl.kernel,
        out_type=jax.ShapeDtypeStruct((3, B, D), jnp.float32),
        mesh=mesh,
        scratch_types=[
            pltpu.VMEM((3 * HIST, D), jnp.float32),
            pltpu.VMEM((NG, GN), jnp.int32),
            pltpu.VMEM((ROWS, D), jnp.float32),
            pltpu.VMEM((3, CB, D), jnp.float32),
            pltpu.SemaphoreType.DMA,
        ],
        compiler_params=pltpu.CompilerParams(use_tc_tiling_on_sc=False),
    )(_sc_partials_body)
    partials = sc_fn(x_r, w_r, table)

    out = pl.pallas_call(
        _tc_finish_body,
        out_shape=jax.ShapeDtypeStruct((B, 3), jnp.float32),
    )(partials[0], partials[1], partials[2], b.reshape(1, 3))
    return out


# speedup vs baseline: 26.9212x; 1.0354x over previous
import functools

import jax
import jax.numpy as jnp
from jax import lax
from jax.experimental import pallas as pl
from jax.experimental.pallas import tpu as pltpu
from jax.experimental.pallas import tpu_sc as plsc

B = 16384
HIST = 50
D = 16

NC = 2
NS = 16
NW = NC * NS
BPW = B // NW
CB = 64
NCHUNK = BPW // CB
ROWS = CB * HIST
NG = HIST // 2


def _sc_partials_body(xt_hbm, w_hbm, tab_hbm, out_hbm, w_v, idx_v, rows_v,
                      o_v, sem):
    cid = lax.axis_index("c")
    sid = lax.axis_index("s")
    wid = sid * NC + cid

    pltpu.sync_copy(w_hbm, w_v)

    def chunk_body(chunk, _):
        base = wid * BPW + chunk * CB
        pltpu.sync_copy(xt_hbm.at[:, pl.ds(base, CB)], idx_v)

        def fire(h, _):
            pltpu.async_copy(tab_hbm.at[idx_v.at[h]],
                             rows_v.at[pl.ds(h * CB, CB)], sem)
            return ()

        lax.fori_loop(0, HIST, fire, (), unroll=False)
        pltpu.make_async_copy(tab_hbm.at[pl.ds(0, ROWS)], rows_v, sem).wait()

        def b_body(bb, _):
            acc0 = jnp.zeros((D,), jnp.float32)
            acc1 = jnp.zeros((D,), jnp.float32)
            acc2 = jnp.zeros((D,), jnp.float32)
            for h in range(HIST):
                r = rows_v[h * CB + bb]
                acc0 = acc0 + r * w_v[h]
                acc1 = acc1 + r * w_v[HIST + h]
                acc2 = acc2 + r * w_v[2 * HIST + h]
            o_v[bb, pl.ds(0, D)] = acc0
            o_v[bb, pl.ds(D, D)] = acc1
            o_v[bb, pl.ds(2 * D, D)] = acc2
            return ()

        lax.fori_loop(0, CB, b_body, (), unroll=False)
        pltpu.sync_copy(o_v, out_hbm.at[pl.ds(base, CB)])
        return ()

    lax.fori_loop(0, NCHUNK, chunk_body, (), unroll=False)


def _tc_finish_body(p_ref, b_ref, o_ref):
    p = p_ref[...]
    s0 = jnp.sum(p[:, 0:D], axis=1, keepdims=True)
    s1 = jnp.sum(p[:, D:2 * D], axis=1, keepdims=True)
    s2 = jnp.sum(p[:, 2 * D:3 * D], axis=1, keepdims=True)
    z = jnp.concatenate([s0, s1, s2], axis=1) + b_ref[...]
    m = jnp.max(z, axis=1, keepdims=True)
    e = jnp.exp(z - m)
    lse = jnp.log(jnp.sum(e, axis=1, keepdims=True))
    o_ref[...] = z - m - lse


@jax.jit
def kernel(x, table, W, b):
    xt = x.astype(jnp.int32).T
    w_r = W.astype(jnp.float32).reshape(3 * HIST, D)

    mesh = plsc.VectorSubcoreMesh(core_axis_name="c", subcore_axis_name="s")
    sc_fn = functools.partial(
        pl.kernel,
        out_type=jax.ShapeDtypeStruct((B, 3 * D), jnp.float32),
        mesh=mesh,
        scratch_types=[
            pltpu.VMEM((3 * HIST, D), jnp.float32),
            pltpu.VMEM((HIST, CB), jnp.int32),
            pltpu.VMEM((ROWS, D), jnp.float32),
            pltpu.VMEM((CB, 3 * D), jnp.float32),
            pltpu.SemaphoreType.DMA,
        ],
        compiler_params=pltpu.CompilerParams(use_tc_tiling_on_sc=False),
    )(_sc_partials_body)
    partials = sc_fn(xt, w_r, table)

    out = pl.pallas_call(
        _tc_finish_body,
        out_shape=jax.ShapeDtypeStruct((B, 3), jnp.float32),
    )(partials, b.reshape(1, 3))
    return out
